# baseline (speedup 1.0000x reference)
import functools

import jax
import jax.numpy as jnp
from jax import lax
from jax.experimental import pallas as pl
from jax.experimental.pallas import tpu as pltpu
from jax.experimental.pallas import tpu_sc as plsc

_VOCAB = 100000
_VOCAB_P = 100096
_NUM_CLASSES = 2
_BATCH = 1024
_SEQ = 200
_SEQ_P = 256
_N_SUBCORES = 16
_ROWS_PER = _BATCH // _N_SUBCORES
_IDS_PER = _ROWS_PER * _SEQ_P
_LANES = 16
_STAGE = _VOCAB_P // _N_SUBCORES


def _bow_body(ids_hbm, w_hbm, out_hbm, shared, ids_v, gat_v, out_v, stage_v,
              sem_w, sem_i, sem_g):
    cls = lax.axis_index("c")
    sub = lax.axis_index("s")
    lane = lax.iota(jnp.int32, _LANES)

    ids_off = pl.multiple_of(sub * _IDS_PER, 8)
    cp_i = pltpu.async_copy(ids_hbm.at[pl.ds(ids_off, _IDS_PER)], ids_v, sem_i)

    w_base = pl.multiple_of(cls * _VOCAB_P, 8)
    sl_off = pl.multiple_of(sub * _STAGE, 8)
    pltpu.async_copy(w_hbm.at[pl.ds(w_base + sl_off, _STAGE)], stage_v,
                     sem_w).wait()

    @pl.when(sub == 0)
    def _patch_pad_entry():
        head = stage_v[pl.ds(0, _LANES)]
        stage_v[pl.ds(0, _LANES)] = jnp.where(lane == 0, jnp.float32(0.0), head)

    pltpu.async_copy(stage_v, shared.at[pl.ds(sl_off, _STAGE)], sem_w).wait()

    cp_i.wait()
    plsc.subcore_barrier()

    pltpu.async_copy(shared.at[ids_v], gat_v, sem_g).wait()

    for g in range(_ROWS_PER // _LANES):
        sums = jnp.zeros((_LANES,), jnp.float32)
        for r16 in range(_LANES):
            row = g * _LANES + r16
            acc = gat_v[pl.ds(row * _SEQ_P, _LANES)]
            for i in range(1, _SEQ_P // _LANES):
                acc = acc + gat_v[pl.ds(row * _SEQ_P + i * _LANES, _LANES)]
            s = jnp.sum(acc)
            sums = jnp.where(lane == r16, s, sums)
        out_v[pl.ds(g * _LANES, _LANES)] = sums

    out_off = pl.multiple_of(cls * _BATCH + sub * _ROWS_PER, 8)
    pltpu.sync_copy(out_v, out_hbm.at[pl.ds(out_off, _ROWS_PER)])


@jax.jit
def _bow_sc(ids_flat, w_flat):
    mesh = plsc.VectorSubcoreMesh(core_axis_name="c", subcore_axis_name="s")
    f = functools.partial(
        pl.kernel,
        mesh=mesh,
        compiler_params=pltpu.CompilerParams(needs_layout_passes=False),
        out_type=jax.ShapeDtypeStruct((_NUM_CLASSES * _BATCH,), jnp.float32),
        scratch_types=[
            pltpu.VMEM_SHARED((_VOCAB_P,), jnp.float32),
            pltpu.VMEM((_IDS_PER,), jnp.int32),
            pltpu.VMEM((_IDS_PER,), jnp.float32),
            pltpu.VMEM((_ROWS_PER,), jnp.float32),
            pltpu.VMEM((_STAGE,), jnp.float32),
            pltpu.SemaphoreType.DMA,
            pltpu.SemaphoreType.DMA,
            pltpu.SemaphoreType.DMA,
        ],
    )(_bow_body)
    return f(ids_flat, w_flat)


def kernel(input_ids, W, b):
    ids = input_ids.astype(jnp.int32)
    ids_p = jnp.pad(ids, ((0, 0), (0, _SEQ_P - _SEQ))).reshape(-1)
    w_flat = jnp.pad(W.astype(jnp.float32),
                     ((0, 0), (0, _VOCAB_P - _VOCAB))).reshape(-1)
    out = _bow_sc(ids_p, w_flat)
    return out.reshape(_NUM_CLASSES, _BATCH).T + b.astype(jnp.float32)


# speedup vs baseline: 1.5471x; 1.5471x over previous
import functools

import jax
import jax.numpy as jnp
from jax import lax
from jax.experimental import pallas as pl
from jax.experimental.pallas import tpu as pltpu
from jax.experimental.pallas import tpu_sc as plsc

_VOCAB = 100000
_NUM_CLASSES = 2
_BATCH = 1024
_SEQ = 200
_SEQ_P = 256
_N_SUBCORES = 16
_ROWS_PER = _BATCH // _N_SUBCORES
_LANES = 16
_GROUPS = _ROWS_PER // _LANES
_HALF = _VOCAB // 2


def _bow_body(ids_hbm, w_hbm, out_hbm, table_v, ids_v, out_v,
              sem_w0, sem_w1, sem_i):
    cls = lax.axis_index("c")
    chunk = lax.axis_index("s")
    rowbase = chunk * _ROWS_PER

    w_base = pl.multiple_of(cls * _VOCAB, 8)
    cp_w0 = pltpu.async_copy(w_hbm.at[pl.ds(w_base, _HALF)],
                             table_v.at[pl.ds(0, _HALF)], sem_w0)
    cp_w1 = pltpu.async_copy(w_hbm.at[pl.ds(w_base + _HALF, _HALF)],
                             table_v.at[pl.ds(_HALF, _HALF)], sem_w1)
    cp_i = pltpu.async_copy(ids_hbm.at[pl.ds(rowbase, _ROWS_PER), :], ids_v,
                            sem_i)
    cp_i.wait()
    cp_w0.wait()
    cp_w1.wait()

    lane = lax.iota(jnp.int32, _LANES)
    head = table_v[pl.ds(0, _LANES)]
    table_v[pl.ds(0, _LANES)] = jnp.where(lane == 0, jnp.float32(0.0), head)

    rows = [g * _LANES + lane for g in range(_GROUPS)]
    zero = jnp.zeros((_LANES,), jnp.float32)

    def step(t, accs):
        tvec = jnp.full((_LANES,), t, jnp.int32)
        ids16 = [plsc.load_gather(ids_v, [rows[g], tvec])
                 for g in range(_GROUPS)]
        vals = [plsc.load_gather(table_v, [ids16[g]]) for g in range(_GROUPS)]
        return tuple(accs[g] + vals[g] for g in range(_GROUPS))

    accs = lax.fori_loop(0, _SEQ, step, (zero,) * _GROUPS)
    for g in range(_GROUPS):
        out_v[pl.ds(g * _LANES, _LANES)] = accs[g]

    out_off = pl.multiple_of(cls * _BATCH + rowbase, 8)
    pltpu.sync_copy(out_v, out_hbm.at[pl.ds(out_off, _ROWS_PER)])


@jax.jit
def _bow_sc(ids_p, w_flat):
    mesh = plsc.VectorSubcoreMesh(core_axis_name="c", subcore_axis_name="s")
    f = functools.partial(
        pl.kernel,
        mesh=mesh,
        compiler_params=pltpu.CompilerParams(needs_layout_passes=False),
        out_type=jax.ShapeDtypeStruct((_NUM_CLASSES * _BATCH,), jnp.float32),
        scratch_types=[
            pltpu.VMEM((_VOCAB,), jnp.float32),
            pltpu.VMEM((_ROWS_PER, _SEQ_P), jnp.int32),
            pltpu.VMEM((_ROWS_PER,), jnp.float32),
            pltpu.SemaphoreType.DMA,
            pltpu.SemaphoreType.DMA,
            pltpu.SemaphoreType.DMA,
        ],
    )(_bow_body)
    return f(ids_p, w_flat)


def kernel(input_ids, W, b):
    ids = input_ids.astype(jnp.int32)
    ids_p = jnp.pad(ids, ((0, 0), (0, _SEQ_P - _SEQ)))
    w_flat = W.astype(jnp.float32).reshape(-1)
    out = _bow_sc(ids_p, w_flat)
    return out.reshape(_NUM_CLASSES, _BATCH).T + b.astype(jnp.float32)


# speedup vs baseline: 1.8863x; 1.2192x over previous
import functools

import jax
import jax.numpy as jnp
from jax import lax
from jax.experimental import pallas as pl
from jax.experimental.pallas import tpu as pltpu
from jax.experimental.pallas import tpu_sc as plsc

_VOCAB = 100000
_NUM_CLASSES = 2
_BATCH = 1024
_SEQ = 200
_N_SUBCORES = 16
_ROWS_PER = _BATCH // _N_SUBCORES
_IDS_PER = _ROWS_PER * _SEQ
_LANES = 16
_GROUPS = _ROWS_PER // _LANES
_N_CHUNKS = 10
_CHUNK = _VOCAB // _N_CHUNKS


def _bow_body(ids_hbm, w_hbm, out_hbm, table_v, ids_v, out_v, sem_w, sem_i):
    cls = lax.axis_index("c")
    sub = lax.axis_index("s")
    rowbase = sub * _ROWS_PER

    ids_off = pl.multiple_of(sub * _IDS_PER, 8)
    cp_i = pltpu.async_copy(ids_hbm.at[pl.ds(ids_off, _IDS_PER)], ids_v, sem_i)

    w_base = pl.multiple_of(cls * _VOCAB, 8)
    start = lax.rem(sub, _N_CHUNKS)
    cps = []
    for k in range(_N_CHUNKS):
        sl = lax.rem(start + k, _N_CHUNKS) * _CHUNK
        sl = pl.multiple_of(sl, 8)
        cps.append(pltpu.async_copy(w_hbm.at[pl.ds(w_base + sl, _CHUNK)],
                                    table_v.at[pl.ds(sl, _CHUNK)], sem_w))
    for cp in cps:
        cp.wait()
    cp_i.wait()

    lane = lax.iota(jnp.int32, _LANES)
    head = table_v[pl.ds(0, _LANES)]
    table_v[pl.ds(0, _LANES)] = jnp.where(lane == 0, jnp.float32(0.0), head)

    bases = [(g * _LANES + lane) * _SEQ for g in range(_GROUPS)]
    zero = jnp.zeros((_LANES,), jnp.float32)

    def step(t, accs):
        ids16 = [plsc.load_gather(ids_v, [bases[g] + t])
                 for g in range(_GROUPS)]
        vals = [plsc.load_gather(table_v, [ids16[g]]) for g in range(_GROUPS)]
        return tuple(accs[g] + vals[g] for g in range(_GROUPS))

    accs = lax.fori_loop(0, _SEQ, step, (zero,) * _GROUPS)
    for g in range(_GROUPS):
        out_v[pl.ds(g * _LANES, _LANES)] = accs[g]

    out_off = pl.multiple_of(cls * _BATCH + rowbase, 8)
    pltpu.sync_copy(out_v, out_hbm.at[pl.ds(out_off, _ROWS_PER)])


@jax.jit
def _bow_sc(ids_flat, w_flat):
    mesh = plsc.VectorSubcoreMesh(core_axis_name="c", subcore_axis_name="s")
    f = functools.partial(
        pl.kernel,
        mesh=mesh,
        compiler_params=pltpu.CompilerParams(needs_layout_passes=False),
        out_type=jax.ShapeDtypeStruct((_NUM_CLASSES * _BATCH,), jnp.float32),
        scratch_types=[
            pltpu.VMEM((_VOCAB,), jnp.float32),
            pltpu.VMEM((_IDS_PER,), jnp.int32),
            pltpu.VMEM((_ROWS_PER,), jnp.float32),
            pltpu.SemaphoreType.DMA,
            pltpu.SemaphoreType.DMA,
        ],
    )(_bow_body)
    return f(ids_flat, w_flat)


def kernel(input_ids, W, b):
    ids_flat = input_ids.astype(jnp.int32).reshape(-1)
    w_flat = W.astype(jnp.float32).reshape(-1)
    out = _bow_sc(ids_flat, w_flat)
    return out.reshape(_NUM_CLASSES, _BATCH).T + b.astype(jnp.float32)
